# Initial kernel scaffold; baseline (speedup 1.0000x reference)
import jax
import jax.numpy as jnp
from jax.experimental import pallas as pl


def kernel(q, k, v):
    raise NotImplementedError("write your pallas kernel here")


# speedup vs baseline: 2.3862x; 2.3862x over previous
import math

import jax
import jax.numpy as jnp
from jax.experimental import pallas as pl
from jax.experimental.pallas import tpu as pltpu

_SEQ = 4096
_HEADS = 16
_DHEAD = 128
_TQ = 128
_W = 384
_NT = _SEQ // _TQ
_SCALE = 1.0 / math.sqrt(_DHEAD)


def _doc_id(x):
    return ((x - 64) * 4162) >> 20


def _flex_attn_kernel(q_ref, k_ref, v_ref, o_ref):
    t = pl.program_id(1)
    q = q_ref[0] * _SCALE
    s = _TQ * jnp.maximum(1, t - 2)

    k1 = k_ref[0, 0:_TQ, :]
    k2 = k_ref[0, pl.ds(s, _W), :]
    kk = jnp.concatenate([k1, k2], axis=0)
    scores = jax.lax.dot_general(
        q, kk, (((1,), (1,)), ((), ())), preferred_element_type=jnp.float32
    )

    r = _TQ * t + jax.lax.broadcasted_iota(jnp.int32, (_TQ, _TQ + _W), 0)
    ci = jax.lax.broadcasted_iota(jnp.int32, (_TQ, _TQ + _W), 1)
    c = jnp.where(ci < _TQ, ci, s - _TQ + ci)
    allowed = (c <= r) & ((r < 64) | (c < 64) | (_doc_id(r) == _doc_id(c)))
    scores = jnp.where(allowed, scores, jnp.float32(-1e30))

    m = jnp.max(scores, axis=1, keepdims=True)
    p = jnp.exp(scores - m)
    l = jnp.sum(p, axis=1, keepdims=True)

    v1 = v_ref[0, 0:_TQ, :]
    v2 = v_ref[0, pl.ds(s, _W), :]
    vv = jnp.concatenate([v1, v2], axis=0)
    o = jax.lax.dot_general(
        p, vv, (((1,), (0,)), ((), ())), preferred_element_type=jnp.float32
    )
    o_ref[0] = o / l


def kernel(q, k, v):
    qh, kh, vh = q[0], k[0], v[0]
    out = pl.pallas_call(
        _flex_attn_kernel,
        grid=(_HEADS, _NT),
        in_specs=[
            pl.BlockSpec((1, _TQ, _DHEAD), lambda h, t: (h, t, 0)),
            pl.BlockSpec((1, _SEQ, _DHEAD), lambda h, t: (h, 0, 0)),
            pl.BlockSpec((1, _SEQ, _DHEAD), lambda h, t: (h, 0, 0)),
        ],
        out_specs=pl.BlockSpec((1, _TQ, _DHEAD), lambda h, t: (h, t, 0)),
        out_shape=jax.ShapeDtypeStruct((_HEADS, _SEQ, _DHEAD), jnp.float32),
        compiler_params=pltpu.CompilerParams(
            dimension_semantics=("arbitrary", "arbitrary")
        ),
    )(qh, kh, vh)
    return out[None]
